# baseline (speedup 1.0000x reference)
import jax
import jax.numpy as jnp
from jax import lax
from jax.experimental import pallas as pl
from jax.experimental.pallas import tpu as pltpu
from jax.experimental.pallas import tpu_sc as plsc

_NUM_FACTORS = 32
_NUM_ROWS = 1000000
_BATCH = 16384
_NUM_CORES = 2
_NUM_SUBCORES = 16
_NW = _NUM_CORES * _NUM_SUBCORES
_RPW = _BATCH // _NW
_CHUNK = 64
_NCH = _RPW // _CHUNK
_LANES = 16
_WIDE = 128
_ROWS_WIDE = _NUM_ROWS * _NUM_FACTORS // _WIDE
_BIAS_ROWS = 7813
_BIAS_PAD = _BIAS_ROWS * _WIDE - _NUM_ROWS


def _mf_body(user_id, item_id, P, Q, ub, ib, out,
             uidx, iidx, uwr, iwr, ubr, ibr, pu, qi, bu, bi, outv,
             sem0, sem1):
    cid = lax.axis_index("c")
    sid = lax.axis_index("s")
    wid = sid * _NUM_CORES + cid
    base = wid * _RPW

    for j in range(_NCH):
        off = base + j * _CHUNK
        pltpu.sync_copy(user_id.at[pl.ds(off, _CHUNK)], uidx.at[j])
        pltpu.sync_copy(item_id.at[pl.ds(off, _CHUNK)], iidx.at[j])

    def stripe_idx(g, carry):
        j = g // (_CHUNK // _LANES)
        c = (g % (_CHUNK // _LANES)) * _LANES
        uv = uidx[j, pl.ds(c, _LANES)]
        iv = iidx[j, pl.ds(c, _LANES)]
        uwr[j, pl.ds(c, _LANES)] = lax.shift_right_logical(uv, 2)
        iwr[j, pl.ds(c, _LANES)] = lax.shift_right_logical(iv, 2)
        ubr[j, pl.ds(c, _LANES)] = lax.shift_right_logical(uv, 7)
        ibr[j, pl.ds(c, _LANES)] = lax.shift_right_logical(iv, 7)
        return carry
    lax.fori_loop(0, _RPW // _LANES, stripe_idx, 0, unroll=True)

    sems = [sem0, sem1]

    def fire(j):
        s = j % 2
        return [pltpu.async_copy(P.at[uwr.at[j]], pu.at[s], sems[s]),
                pltpu.async_copy(Q.at[iwr.at[j]], qi.at[s], sems[s]),
                pltpu.async_copy(ub.at[ubr.at[j]], bu.at[s], sems[s]),
                pltpu.async_copy(ib.at[ibr.at[j]], bi.at[s], sems[s])]

    inflight = {0: fire(0), 1: fire(1)}

    lane_iota = lax.iota(jnp.int32, _LANES)

    def compute_chunk(j):
        s = j % 2
        s_splat = jnp.full((_LANES,), s, jnp.int32)

        def group(g, carry):
            c = g * _LANES
            rows = lane_iota + c
            ul = uidx[j, pl.ds(c, _LANES)]
            il = iidx[j, pl.ds(c, _LANES)]
            ubase = (ul & 3) * _NUM_FACTORS
            ibase = (il & 3) * _NUM_FACTORS
            acc = (plsc.load_gather(bu, [s_splat, rows, ul & 127])
                   + plsc.load_gather(bi, [s_splat, rows, il & 127]))
            for f in range(_NUM_FACTORS):
                acc = acc + (plsc.load_gather(pu, [s_splat, rows, ubase + f])
                             * plsc.load_gather(qi, [s_splat, rows, ibase + f]))
            outv[pl.ds(j * _CHUNK + c, _LANES)] = acc
            return carry

        lax.fori_loop(0, _CHUNK // _LANES, group, 0)

    for j in range(_NCH):
        for cp in inflight.pop(j):
            cp.wait()
        compute_chunk(j)
        if j + 2 < _NCH:
            inflight[j + 2] = fire(j + 2)

    pltpu.sync_copy(outv, out.at[pl.ds(base, _RPW)])


@jax.jit
def _mf(user_id, item_id, P, Q, ub, ib):
    mesh = plsc.VectorSubcoreMesh(core_axis_name="c", subcore_axis_name="s")
    kern = pl.kernel(
        _mf_body,
        out_type=jax.ShapeDtypeStruct((_BATCH,), jnp.float32),
        mesh=mesh,
        compiler_params=pltpu.CompilerParams(needs_layout_passes=False),
        scratch_types=[
            pltpu.VMEM((_NCH, _CHUNK), jnp.int32),
            pltpu.VMEM((_NCH, _CHUNK), jnp.int32),
            pltpu.VMEM((_NCH, _CHUNK), jnp.int32),
            pltpu.VMEM((_NCH, _CHUNK), jnp.int32),
            pltpu.VMEM((_NCH, _CHUNK), jnp.int32),
            pltpu.VMEM((_NCH, _CHUNK), jnp.int32),
            pltpu.VMEM((2, _CHUNK, _WIDE), jnp.float32),
            pltpu.VMEM((2, _CHUNK, _WIDE), jnp.float32),
            pltpu.VMEM((2, _CHUNK, _WIDE), jnp.float32),
            pltpu.VMEM((2, _CHUNK, _WIDE), jnp.float32),
            pltpu.VMEM((_RPW,), jnp.float32),
            pltpu.SemaphoreType.DMA,
            pltpu.SemaphoreType.DMA,
        ],
    )
    return kern(user_id, item_id, P, Q, ub, ib)


def kernel(user_id, item_id, P, Q, user_bias, item_bias):
    P4 = P.reshape(_ROWS_WIDE, _WIDE)
    Q4 = Q.reshape(_ROWS_WIDE, _WIDE)
    pad = jnp.zeros((_BIAS_PAD,), jnp.float32)
    ub = jnp.concatenate([user_bias.reshape(-1), pad]).reshape(
        _BIAS_ROWS, _WIDE)
    ib = jnp.concatenate([item_bias.reshape(-1), pad]).reshape(
        _BIAS_ROWS, _WIDE)
    return _mf(user_id, item_id, P4, Q4, ub, ib)


# speedup vs baseline: 1.0107x; 1.0107x over previous
import jax
import jax.numpy as jnp
from jax import lax
from jax.experimental import pallas as pl
from jax.experimental.pallas import tpu as pltpu
from jax.experimental.pallas import tpu_sc as plsc

_NUM_FACTORS = 32
_NUM_ROWS = 1000000
_BATCH = 16384
_NUM_CORES = 2
_NUM_SUBCORES = 16
_NW = _NUM_CORES * _NUM_SUBCORES
_RPW = _BATCH // _NW
_CHUNK = 64
_NCH = _RPW // _CHUNK
_LANES = 16
_WIDE = 128
_ROWS_WIDE = _NUM_ROWS * _NUM_FACTORS // _WIDE
_BIAS_ROWS = 7813
_BIAS_PAD = _BIAS_ROWS * _WIDE - _NUM_ROWS


def _mf_body(user_id, item_id, P, Q, ub, ib, out,
             uidx, iidx, uwr, iwr, ubr, ibr, pu, qi, bu, bi, outv,
             sem0, sem1):
    cid = lax.axis_index("c")
    sid = lax.axis_index("s")
    wid = sid * _NUM_CORES + cid
    base = wid * _RPW

    pltpu.sync_copy(user_id.at[pl.ds(base, _RPW)], uidx)
    pltpu.sync_copy(item_id.at[pl.ds(base, _RPW)], iidx)

    def stripe_idx(g, carry):
        c = g * _LANES
        uv = uidx[pl.ds(c, _LANES)]
        iv = iidx[pl.ds(c, _LANES)]
        uwr[pl.ds(c, _LANES)] = lax.shift_right_logical(uv, 2)
        iwr[pl.ds(c, _LANES)] = lax.shift_right_logical(iv, 2)
        ubr[pl.ds(c, _LANES)] = lax.shift_right_logical(uv, 7)
        ibr[pl.ds(c, _LANES)] = lax.shift_right_logical(iv, 7)
        return carry
    lax.fori_loop(0, _RPW // _LANES, stripe_idx, 0)

    sems = [sem0, sem1]
    bufs = [(pu.at[0], qi.at[0], bu.at[0], bi.at[0]),
            (pu.at[1], qi.at[1], bu.at[1], bi.at[1])]

    def descriptors(j, b):
        o = j * _CHUNK
        pub, qib, bub, bib = bufs[b]
        s = sems[b]
        return [pltpu.make_async_copy(P.at[uwr.at[pl.ds(o, _CHUNK)]], pub, s),
                pltpu.make_async_copy(Q.at[iwr.at[pl.ds(o, _CHUNK)]], qib, s),
                pltpu.make_async_copy(ub.at[ubr.at[pl.ds(o, _CHUNK)]], bub, s),
                pltpu.make_async_copy(ib.at[ibr.at[pl.ds(o, _CHUNK)]], bib, s)]

    def fire(j, b):
        for d in descriptors(j, b):
            d.start()

    def drain(j, b):
        for d in descriptors(j, b):
            d.wait()

    fire(0, 0)
    fire(1, 1)

    lane_iota = lax.iota(jnp.int32, _LANES)

    def round_body(g, carry):
        for b in range(2):
            j = 2 * g + b
            drain(j, b)
            pub, qib, bub, bib = bufs[b]

            def group(g2, carry2):
                c = g2 * _LANES
                rows = lane_iota + c
                ul = uidx[pl.ds(j * _CHUNK + c, _LANES)]
                il = iidx[pl.ds(j * _CHUNK + c, _LANES)]
                ubase = (ul & 3) * _NUM_FACTORS
                ibase = (il & 3) * _NUM_FACTORS
                acc = (plsc.load_gather(bub, [rows, ul & 127])
                       + plsc.load_gather(bib, [rows, il & 127]))
                for f in range(_NUM_FACTORS):
                    acc = acc + (plsc.load_gather(pub, [rows, ubase + f])
                                 * plsc.load_gather(qib, [rows, ibase + f]))
                outv[pl.ds(j * _CHUNK + c, _LANES)] = acc
                return carry2

            lax.fori_loop(0, _CHUNK // _LANES, group, 0)
            pl.when(g < _NCH // 2 - 1)(lambda: fire(j + 2, b))
        return carry

    lax.fori_loop(0, _NCH // 2, round_body, 0)

    pltpu.sync_copy(outv, out.at[pl.ds(base, _RPW)])


@jax.jit
def _mf(user_id, item_id, P, Q, ub, ib):
    mesh = plsc.VectorSubcoreMesh(core_axis_name="c", subcore_axis_name="s")
    kern = pl.kernel(
        _mf_body,
        out_type=jax.ShapeDtypeStruct((_BATCH,), jnp.float32),
        mesh=mesh,
        compiler_params=pltpu.CompilerParams(needs_layout_passes=False),
        scratch_types=[
            pltpu.VMEM((_RPW,), jnp.int32),
            pltpu.VMEM((_RPW,), jnp.int32),
            pltpu.VMEM((_RPW,), jnp.int32),
            pltpu.VMEM((_RPW,), jnp.int32),
            pltpu.VMEM((_RPW,), jnp.int32),
            pltpu.VMEM((_RPW,), jnp.int32),
            pltpu.VMEM((2, _CHUNK, _WIDE), jnp.float32),
            pltpu.VMEM((2, _CHUNK, _WIDE), jnp.float32),
            pltpu.VMEM((2, _CHUNK, _WIDE), jnp.float32),
            pltpu.VMEM((2, _CHUNK, _WIDE), jnp.float32),
            pltpu.VMEM((_RPW,), jnp.float32),
            pltpu.SemaphoreType.DMA,
            pltpu.SemaphoreType.DMA,
        ],
    )
    return kern(user_id, item_id, P, Q, ub, ib)


def kernel(user_id, item_id, P, Q, user_bias, item_bias):
    P4 = P.reshape(_ROWS_WIDE, _WIDE)
    Q4 = Q.reshape(_ROWS_WIDE, _WIDE)
    pad = jnp.zeros((_BIAS_PAD,), jnp.float32)
    ub = jnp.concatenate([user_bias.reshape(-1), pad]).reshape(
        _BIAS_ROWS, _WIDE)
    ib = jnp.concatenate([item_bias.reshape(-1), pad]).reshape(
        _BIAS_ROWS, _WIDE)
    return _mf(user_id, item_id, P4, Q4, ub, ib)
